# baseline (speedup 1.0000x reference)
import functools

import jax
import jax.numpy as jnp
from jax import lax
from jax.experimental import pallas as pl
from jax.experimental.pallas import tpu as pltpu
from jax.experimental.pallas import tpu_sc as plsc

EMBED = 64
ROW_W = 128
CHUNKB = 256
NBUF = 2
BTILES = CHUNKB // 128


@functools.lru_cache(maxsize=None)
def _make_gather(batch: int, seq: int):
    info = plsc.get_sparse_core_info()
    nw = info.num_cores * info.num_subcores
    n_total = batch * seq
    per_w = n_total // nw
    n_chunks = per_w // CHUNKB
    q_per_l = batch // CHUNKB
    assert per_w * nw == n_total and n_chunks * CHUNKB == per_w
    assert n_chunks % NBUF == 0 and n_chunks // NBUF >= 3
    n_rounds = n_chunks // NBUF
    mesh = plsc.VectorSubcoreMesh(core_axis_name="c", subcore_axis_name="s")

    @functools.partial(
        pl.kernel,
        mesh=mesh,
        out_type=jax.ShapeDtypeStruct((seq, 8, batch // 128, 8, 128), jnp.float32),
        scratch_types=[
            pltpu.VMEM((NBUF, CHUNKB), jnp.int32),
            pltpu.VMEM((NBUF, CHUNKB, ROW_W), jnp.float32),
            pltpu.VMEM((NBUF, 1, 8, BTILES, 8, 128), jnp.float32),
            pltpu.SemaphoreType.DMA((NBUF,)),
            pltpu.SemaphoreType.DMA((NBUF,)),
            pltpu.SemaphoreType.DMA((NBUF,)),
        ],
        compiler_params=pltpu.CompilerParams(
            use_tc_tiling_on_sc=False, needs_layout_passes=False),
    )
    def gather_kernel(idx_hbm, ptab_hbm, out_hbm, ibuf, grows, tiles,
                      gsem, ssem, isem):
        wid = lax.axis_index("s") * info.num_cores + lax.axis_index("c")
        base = wid * per_w
        lane = lax.broadcasted_iota(jnp.int32, (16,), 0)

        def idx_start(clocal, nb):
            src = idx_hbm.at[pl.ds(base + clocal * CHUNKB, CHUNKB)]
            pltpu.async_copy(src, ibuf.at[nb], isem.at[nb])

        def idx_wait(clocal, nb):
            src = idx_hbm.at[pl.ds(base + clocal * CHUNKB, CHUNKB)]
            pltpu.make_async_copy(src, ibuf.at[nb], isem.at[nb]).wait()

        def gather_start(clocal, nb):
            pltpu.async_copy(
                ptab_hbm.at[ibuf.at[nb]], grows.at[nb], gsem.at[nb])

        def gather_wait(clocal, nb):
            pltpu.make_async_copy(
                ptab_hbm.at[ibuf.at[nb]], grows.at[nb], gsem.at[nb]).wait()

        def out_slice(clocal):
            cglob = wid * n_chunks + clocal
            l = cglob // q_per_l
            bt0 = (cglob % q_per_l) * BTILES
            return out_hbm.at[pl.ds(l, 1), :, pl.ds(bt0, BTILES)]

        def store_start(clocal, nb):
            pltpu.async_copy(tiles.at[nb], out_slice(clocal), ssem.at[nb])

        def store_wait(clocal, nb):
            pltpu.make_async_copy(
                tiles.at[nb], out_slice(clocal), ssem.at[nb]).wait()

        def transpose(nb):
            g = grows.at[nb]
            for et in range(8):
                for bt in range(BTILES):
                    for el in range(8):
                        col = lane * 0 + (et * 8 + el)
                        for k in range(8):
                            rows = lane + (bt * 128 + k * 16)
                            v = plsc.load_gather(g, [rows, col])
                            tiles[nb, 0, et, bt, el, pl.ds(k * 16, 16)] = v

        for nb in range(NBUF):
            idx_start(nb, nb)
        for nb in range(NBUF):
            idx_wait(nb, nb)
            gather_start(nb, nb)

        def body(g, carry):
            c0 = g * NBUF
            for nb in range(NBUF):
                c = c0 + nb
                gather_wait(c, nb)

                @pl.when(g < n_rounds - 1)
                def _():
                    idx_start(c + NBUF, nb)

                @pl.when(g > 0)
                def _():
                    store_wait(c - NBUF, nb)

                transpose(nb)
                store_start(c, nb)

                @pl.when(g < n_rounds - 1)
                def _():
                    idx_wait(c + NBUF, nb)
                    gather_start(c + NBUF, nb)

            return carry

        lax.fori_loop(0, n_rounds, body, 0)

        cf = (n_rounds - 1) * NBUF
        for nb in range(NBUF):
            store_wait(cf + nb, nb)

    return gather_kernel


def kernel(x, table):
    b, l = x.shape
    flat = x.T.reshape(b * l).astype(jnp.int32)
    padded = jnp.pad(table, ((0, 0), (0, ROW_W - EMBED)))
    out5d = _make_gather(b, l)(flat, padded)
    return out5d.transpose(2, 4, 0, 1, 3).reshape(b, l, EMBED)
